# baseline (speedup 1.0000x reference)
import functools

import jax
import jax.numpy as jnp
from jax import lax
from jax.experimental import pallas as pl
from jax.experimental.pallas import tpu as pltpu
from jax.experimental.pallas import tpu_sc as plsc

VOCAB = 1000000
D = 16
P = 16
NC = 2
NS = 16
NW = NC * NS

_SIN_C = (3.1415926409395194, -5.16771227680099, 2.550158280611899,
          -0.5992355764431792, 0.08207129109386657, -0.0072673205351405645,
          0.00039296507712438533)
_COS_C = (0.9999999999193584, -4.9348021895543805, 4.0587118821364,
          -1.3352607094469748, 0.23532212897209104, -0.025787854658556375,
          0.0019059119592104157, -8.916973064498901e-05)


def _horner(w, coeffs):
    r = jnp.full((D,), coeffs[-1], dtype=jnp.float32)
    for c in coeffs[-2::-1]:
        r = r * w + c
    return r


def _make_sc_kernel(n_tokens):
    n_per = n_tokens // NW
    mesh = plsc.VectorSubcoreMesh(core_axis_name="c", subcore_axis_name="s",
                                  num_cores=NC, num_subcores=NS)

    @functools.partial(
        pl.kernel,
        out_type=(jax.ShapeDtypeStruct((n_tokens, D), jnp.float32),
                  jax.ShapeDtypeStruct((n_tokens, D), jnp.float32)),
        mesh=mesh,
        scratch_types=[
            pltpu.VMEM((n_per,), jnp.int32),
            pltpu.VMEM((n_per, D), jnp.float32),
            pltpu.VMEM((n_per, D), jnp.float32),
            pltpu.VMEM((n_per, D), jnp.float32),
            pltpu.SemaphoreType.DMA,
        ],
        compiler_params=pltpu.CompilerParams(use_tc_tiling_on_sc=False),
    )
    def sc_fn(tok_hbm, tab_hbm, re_hbm, im_hbm, idx_v, rows_v, re_v, im_v,
              sem):
        wid = lax.axis_index("s") * NC + lax.axis_index("c")
        base = wid * n_per
        pltpu.sync_copy(tok_hbm.at[pl.ds(base, n_per)], idx_v)
        pltpu.async_copy(tab_hbm.at[idx_v], rows_v, sem).wait()

        def body(i, carry):
            v = rows_v[i]
            a = jnp.abs(v)
            e = jnp.exp(a * -2.0)
            u = jnp.sign(v) * ((1.0 - e) / (1.0 + e))
            w = u * u
            im_v[i] = u * _horner(w, _SIN_C)
            re_v[i] = _horner(w, _COS_C)
            return carry

        lax.fori_loop(0, n_per, body, 0)
        pltpu.sync_copy(re_v, re_hbm.at[pl.ds(base, n_per)])
        pltpu.sync_copy(im_v, im_hbm.at[pl.ds(base, n_per)])

    return sc_fn


def kernel(token_ids, embedding_table):
    b, l = token_ids.shape
    n = b * l
    tok = token_ids.reshape(n).astype(jnp.int32)
    re, im = _make_sc_kernel(n)(tok, embedding_table)
    base = lax.complex(re, im).reshape(b, l, 1, D)
    return jnp.broadcast_to(base, (b, l, P, D))
